# Initial kernel scaffold; baseline (speedup 1.0000x reference)
import jax
import jax.numpy as jnp
from jax.experimental import pallas as pl


def kernel(x, emb):
    raise NotImplementedError("write your pallas kernel here")


# speedup vs baseline: 3.0336x; 3.0336x over previous
import jax
import jax.numpy as jnp
from jax.experimental import pallas as pl


def _scale_copy(emb_ref, o_ref, *, scale):
    o_ref[...] = emb_ref[...] * scale


def kernel(x, emb):
    seq_len = x.shape[1]
    dim = emb.shape[1]
    scale = dim ** (-0.5)
    rows_per_block = 1024
    assert seq_len % rows_per_block == 0
    import functools
    return pl.pallas_call(
        functools.partial(_scale_copy, scale=scale),
        grid=(seq_len // rows_per_block,),
        in_specs=[pl.BlockSpec((rows_per_block, dim), lambda i: (i, 0))],
        out_specs=pl.BlockSpec((rows_per_block, dim), lambda i: (i, 0)),
        out_shape=jax.ShapeDtypeStruct((seq_len, dim), emb.dtype),
    )(emb[:seq_len])


# speedup vs baseline: 3.2647x; 1.0762x over previous
import jax
import jax.numpy as jnp
from jax.experimental import pallas as pl


def _scale_copy(emb_ref, o_ref, *, scale):
    o_ref[...] = emb_ref[...] * scale


def kernel(x, emb):
    seq_len = x.shape[1]
    dim = emb.shape[1]
    scale = dim ** (-0.5)
    rows_per_block = 2048
    assert seq_len % rows_per_block == 0
    import functools
    return pl.pallas_call(
        functools.partial(_scale_copy, scale=scale),
        grid=(seq_len // rows_per_block,),
        in_specs=[pl.BlockSpec((rows_per_block, dim), lambda i: (i, 0))],
        out_specs=pl.BlockSpec((rows_per_block, dim), lambda i: (i, 0)),
        out_shape=jax.ShapeDtypeStruct((seq_len, dim), emb.dtype),
    )(emb[:seq_len])
